# Initial kernel scaffold; baseline (speedup 1.0000x reference)
import jax
import jax.numpy as jnp
from jax.experimental import pallas as pl


def kernel(x, edge_index, emb, W1, b1, W2, b2, W3, b3, Wmu, bmu, Wls, bls):
    raise NotImplementedError("write your pallas kernel here")


# speedup vs baseline: 5.1014x; 5.1014x over previous
import functools

import jax
import jax.numpy as jnp
from jax import lax
from jax.experimental import pallas as pl
from jax.experimental.pallas import tpu as pltpu
from jax.experimental.pallas import tpu_sc as plsc

N = 10000
NP = 10240
E = 320000
EP = 327680
VOCAB = 3000
VOCABP = 3072
NC = 2
NS = 16
WCH = 80
CW = 128
RPT = NP // NS
QW = 80
NPW = NP // (NC * NS)
LW = 128

_MESH = dict(core_axis_name="c", subcore_axis_name="s", num_cores=NC,
             num_subcores=NS)


def _sc_prep(dst_p, nidx, embW, ones_rows, zeros_rows):
    @functools.partial(
        pl.kernel,
        out_type=[jax.ShapeDtypeStruct((NC, NP, LW), jnp.float32),
                  jax.ShapeDtypeStruct((NP, LW), jnp.float32)],
        mesh=plsc.VectorSubcoreMesh(**_MESH),
        scratch_types=[
            pltpu.VMEM((CW,), jnp.int32),
            pltpu.VMEM((QW,), jnp.int32),
            pltpu.VMEM((CW, LW), jnp.float32),
            pltpu.VMEM((QW, LW), jnp.float32),
            pltpu.VMEM_SHARED((NP, LW), jnp.float32),
        ],
    )
    def body(dst_hbm, nidx_hbm, embW_hbm, ones_hbm, zeros_hbm,
             deg_out, emb_out, dst_v, nidx_v, ones_v, rows_v, acc):
        cid = lax.axis_index("c")
        sid = lax.axis_index("s")
        wid = sid * NC + cid

        pltpu.sync_copy(zeros_hbm, ones_v)

        @pl.loop(0, RPT // CW)
        def _(i):
            pltpu.sync_copy(ones_v, acc.at[pl.ds(sid * RPT + i * CW, CW)])

        pltpu.sync_copy(ones_hbm, ones_v)
        plsc.subcore_barrier()

        ebase = wid * WCH * CW

        @pl.loop(0, WCH)
        def _(b):
            pltpu.sync_copy(dst_hbm.at[pl.ds(ebase + b * CW, CW)], dst_v)
            pltpu.sync_copy(ones_v, acc.at[dst_v], add=True)

        nbase = wid * NPW

        @pl.loop(0, NPW // QW)
        def _(j):
            pltpu.sync_copy(nidx_hbm.at[pl.ds(nbase + j * QW, QW)], nidx_v)
            pltpu.sync_copy(embW_hbm.at[nidx_v], rows_v)
            pltpu.sync_copy(rows_v, emb_out.at[pl.ds(nbase + j * QW, QW)])

        plsc.subcore_barrier()
        pltpu.sync_copy(acc.at[pl.ds(sid * RPT, RPT)],
                        deg_out.at[cid, pl.ds(sid * RPT, RPT)])

    return body(dst_p, nidx, embW, ones_rows, zeros_rows)


def _sc_agg(src_p, dst_p, g, zeros_rows):
    @functools.partial(
        pl.kernel,
        out_type=jax.ShapeDtypeStruct((NC, NP, LW), jnp.float32),
        mesh=plsc.VectorSubcoreMesh(**_MESH),
        scratch_types=[
            pltpu.VMEM((CW,), jnp.int32),
            pltpu.VMEM((CW,), jnp.int32),
            pltpu.VMEM((CW, LW), jnp.float32),
            pltpu.VMEM_SHARED((NP, LW), jnp.float32),
        ],
    )
    def body(src_hbm, dst_hbm, g_hbm, zeros_hbm, out,
             src_v, dst_v, rows_v, acc):
        cid = lax.axis_index("c")
        sid = lax.axis_index("s")
        wid = sid * NC + cid

        pltpu.sync_copy(zeros_hbm, rows_v)

        @pl.loop(0, RPT // CW)
        def _(i):
            pltpu.sync_copy(rows_v, acc.at[pl.ds(sid * RPT + i * CW, CW)])

        plsc.subcore_barrier()

        ebase = wid * WCH * CW

        @pl.loop(0, WCH)
        def _(b):
            pltpu.sync_copy(src_hbm.at[pl.ds(ebase + b * CW, CW)], src_v)
            pltpu.sync_copy(dst_hbm.at[pl.ds(ebase + b * CW, CW)], dst_v)
            pltpu.sync_copy(g_hbm.at[src_v], rows_v)
            pltpu.sync_copy(rows_v, acc.at[dst_v], add=True)

        plsc.subcore_barrier()
        pltpu.sync_copy(acc.at[pl.ds(sid * RPT, RPT)],
                        out.at[cid, pl.ds(sid * RPT, RPT)])

    return body(src_p, dst_p, g, zeros_rows)


def _pad_cols(z):
    return jnp.concatenate(
        [z, jnp.zeros((z.shape[0], LW - z.shape[1]), jnp.float32)], axis=1)


def _tc_pre(x, W1a, emb, W1b):
    def body(x_ref, wa_ref, emb_ref, wb_ref, xw_ref, embW_ref):
        xw_ref[...] = jnp.dot(x_ref[...], wa_ref[...],
                              preferred_element_type=jnp.float32)
        embW_ref[...] = _pad_cols(jnp.dot(emb_ref[...], wb_ref[...],
                                          preferred_element_type=jnp.float32))

    return pl.pallas_call(
        body,
        out_shape=[jax.ShapeDtypeStruct((N, 32), jnp.float32),
                   jax.ShapeDtypeStruct((VOCABP, LW), jnp.float32)],
    )(x, W1a, emb, W1b)


def _tc_l1(deg_parts, xw_pad, embrows):
    def body(dp_ref, xw_ref, er_ref, g1_ref, dinv_ref):
        deg = dp_ref[0, :, 0:1] + dp_ref[1, :, 0:1] + 1.0
        dinv = jnp.broadcast_to(lax.rsqrt(deg), (NP, 32))
        dinv_ref[...] = dinv
        z = xw_ref[...] + er_ref[:, :32]
        g1_ref[...] = _pad_cols(z * dinv)

    return pl.pallas_call(
        body,
        out_shape=[jax.ShapeDtypeStruct((NP, LW), jnp.float32),
                   jax.ShapeDtypeStruct((NP, 32), jnp.float32)],
    )(deg_parts, xw_pad, embrows)


def _tc_layer(s_parts, g, dinv32, b, W):
    def body(s_ref, g_ref, di_ref, b_ref, w_ref, out_ref):
        s = s_ref[0, :, :32] + s_ref[1, :, :32] + g_ref[:, :32]
        h = jnp.maximum(s * di_ref[...] + b_ref[...], 0.0)
        z = jnp.dot(h, w_ref[...],
                    preferred_element_type=jnp.float32)
        out_ref[...] = _pad_cols(z * di_ref[:, :z.shape[1]])

    return pl.pallas_call(
        body,
        out_shape=jax.ShapeDtypeStruct((NP, LW), jnp.float32),
    )(s_parts, g, dinv32, b, W)


def _tc_mu(s_parts, g, dinv32, b):
    def body(s_ref, g_ref, di_ref, b_ref, out_ref):
        s = s_ref[0, :, :16] + s_ref[1, :, :16] + g_ref[:, :16]
        out_ref[...] = s * di_ref[:, :16] + b_ref[...]

    return pl.pallas_call(
        body,
        out_shape=jax.ShapeDtypeStruct((NP, 16), jnp.float32),
    )(s_parts, g, dinv32, b)


def kernel(x, edge_index, emb, W1, b1, W2, b2, W3, b3, Wmu, bmu, Wls, bls):
    idx = x[:, -1].astype(jnp.int32)
    src = edge_index[0].astype(jnp.int32)
    dst = edge_index[1].astype(jnp.int32)
    pad = jnp.full((EP - E,), N, jnp.int32)
    src_p = jnp.concatenate([src, pad])
    dst_p = jnp.concatenate([dst, pad])
    nidx = jnp.concatenate([idx, jnp.zeros((NP - N,), jnp.int32)])

    W1a = jnp.pad(W1[:127], ((0, 1), (0, 0)))
    W1b = W1[127:]
    emb_p = jnp.pad(emb, ((0, VOCABP - VOCAB), (0, 0)))
    ones_rows = jnp.ones((CW, LW), jnp.float32)
    zeros_rows = jnp.zeros((CW, LW), jnp.float32)

    xw, embW = _tc_pre(x, W1a, emb_p, W1b)
    xw_pad = jnp.pad(xw, ((0, NP - N), (0, 0)))

    deg_parts, embrows = _sc_prep(dst_p, nidx, embW, ones_rows, zeros_rows)
    g1, dinv32 = _tc_l1(deg_parts, xw_pad, embrows)

    s1 = _sc_agg(src_p, dst_p, g1, zeros_rows)
    g2 = _tc_layer(s1, g1, dinv32, b1.reshape(1, 32), W2)
    s2 = _sc_agg(src_p, dst_p, g2, zeros_rows)
    g3 = _tc_layer(s2, g2, dinv32, b2.reshape(1, 32), W3)
    s3 = _sc_agg(src_p, dst_p, g3, zeros_rows)
    g4 = _tc_layer(s3, g3, dinv32, b3.reshape(1, 32), Wmu)
    s4 = _sc_agg(src_p, dst_p, g4, zeros_rows)
    mu = _tc_mu(s4, g4, dinv32, bmu.reshape(1, 16))
    return mu[:N]
